# baseline (speedup 1.0000x reference)
import functools

import jax
import jax.numpy as jnp
from jax import lax
from jax.experimental import pallas as pl
from jax.experimental.pallas import tpu as pltpu
from jax.experimental.pallas import tpu_sc as plsc

_VOCAB = 1000000
_EMBED = 64
_B = 4096
_N = 200

_NC = 2
_NS = 16
_NW = _NC * _NS
_WPG = 4
_NG = _NW // _WPG
_CPG = _N // _NG
_QB = _B // _WPG
_CH = 128
_JC = _QB // _CH
_ET = _EMBED // 8
_BT = _B // _CH

_mesh = plsc.VectorSubcoreMesh(core_axis_name="c", subcore_axis_name="s")


@functools.partial(
    pl.kernel,
    mesh=_mesh,
    compiler_params=pltpu.CompilerParams(
        use_tc_tiling_on_sc=False, needs_layout_passes=False
    ),
    out_type=jax.ShapeDtypeStruct((_N, _ET, _BT, 8 * _CH), jnp.float32),
    scratch_types=[
        pltpu.VMEM((_CPG, _JC, _CH), jnp.int32),
        pltpu.VMEM((2, _CH, _EMBED), jnp.float32),
        pltpu.VMEM((_ET, 8 * _CH), jnp.float32),
        pltpu.VMEM((_ET, 8 * _CH), jnp.float32),
        pltpu.VMEM((_N, _EMBED), jnp.float32),
        pltpu.SemaphoreType.DMA,
        pltpu.SemaphoreType.DMA,
        pltpu.SemaphoreType.DMA,
        pltpu.SemaphoreType.DMA,
    ],
)
def _embed_sc(xT_hbm, tok_hbm, pos_hbm, out_hbm, idx_v, grows_v, obuf0, obuf1,
              pos_v, gsem0, gsem1, osem0, osem1):
    cid = lax.axis_index("c")
    sid = lax.axis_index("s")
    wid = sid * _NC + cid
    grp = wid // _WPG
    sub = wid % _WPG
    n0 = grp * _CPG
    bt0 = sub * _JC

    pltpu.sync_copy(pos_hbm, pos_v)
    pltpu.sync_copy(xT_hbm.at[pl.ds(n0, _CPG), pl.ds(bt0, _JC)], idx_v)

    iota16 = lax.iota(jnp.int32, 16)
    er = [(iota16 + 16 * d) >> 3 for d in range(4)]
    wib = [((iota16 + 16 * d) & 7) * _CH for d in range(4)]

    def gsem(b):
        return gsem0 if b == 0 else gsem1

    def osem(b):
        return osem0 if b == 0 else osem1

    def obuf(b):
        return obuf0 if b == 0 else obuf1

    def fire_gather(t, j):
        b = j % 2
        pltpu.async_copy(tok_hbm.at[idx_v.at[t, j]], grows_v.at[b], gsem(b))

    def out_slice(t, j):
        return out_hbm.at[n0 + t, pl.ds(0, _ET), bt0 + j]

    def slot(t, j, do_outwait, do_fire):
        b = j % 2
        n = n0 + t
        pltpu.make_async_copy(
            tok_hbm.at[idx_v.at[t, j]], grows_v.at[b], gsem(b)
        ).wait()
        if do_outwait:
            j3 = (j - 2) % _JC
            t3 = t - 1 if j < 2 else t
            pltpu.make_async_copy(obuf(b), out_slice(t3, j3), osem(b)).wait()
        prow = [pos_v[n, pl.ds(16 * d, 16)] for d in range(4)]

        def add_body(i, c):
            for ii in range(4):
                row = i * 4 + ii
                for d in range(4):
                    vals = grows_v[b, row, pl.ds(16 * d, 16)] + prow[d]
                    plsc.store_scatter(obuf(b), [er[d], wib[d] + row], vals)
            return c

        lax.fori_loop(0, _CH // 4, add_body, 0)
        pltpu.async_copy(obuf(b), out_slice(t, j), osem(b))
        if do_fire:
            j2 = (j + 2) % _JC
            t2 = t + 1 if j >= _JC - 2 else t
            fire_gather(t2, j2)

    fire_gather(0, 0)
    fire_gather(0, 1)

    for j in range(_JC):
        slot(0, j, do_outwait=(j >= 2), do_fire=True)

    def t_body(t, carry):
        for j in range(_JC):
            slot(t, j, do_outwait=True, do_fire=True)
        return carry

    lax.fori_loop(1, _CPG - 1, t_body, 0)

    for j in range(_JC):
        slot(_CPG - 1, j, do_outwait=True, do_fire=(j < _JC - 2))

    pltpu.make_async_copy(obuf0, out_slice(_CPG - 1, _JC - 2), osem0).wait()
    pltpu.make_async_copy(obuf1, out_slice(_CPG - 1, _JC - 1), osem1).wait()


def kernel(x, tok_table, pos_table):
    xT3 = jnp.transpose(x.astype(jnp.int32)).reshape(_N, _BT, _CH)
    tiled = _embed_sc(xT3, tok_table, pos_table)
    t5 = tiled.reshape(_N, _ET, _BT, 8, _CH)
    return jnp.transpose(t5, (2, 4, 0, 1, 3)).reshape(_B, _N, _EMBED)


# speedup vs baseline: 1.5157x; 1.5157x over previous
import functools

import jax
import jax.numpy as jnp
from jax import lax
from jax.experimental import pallas as pl
from jax.experimental.pallas import tpu as pltpu
from jax.experimental.pallas import tpu_sc as plsc

_VOCAB = 1000000
_EMBED = 64
_B = 4096
_N = 200

_NC = 2
_NS = 16
_NW = _NC * _NS
_WPG = 4
_NG = _NW // _WPG
_CPG = _N // _NG
_QB = _B // _WPG
_CH = 128
_JC = _QB // _CH
_ET = _EMBED // 8
_BT = _B // _CH

_mesh = plsc.VectorSubcoreMesh(core_axis_name="c", subcore_axis_name="s")


@functools.partial(
    pl.kernel,
    mesh=_mesh,
    compiler_params=pltpu.CompilerParams(
        use_tc_tiling_on_sc=False, needs_layout_passes=False
    ),
    out_type=jax.ShapeDtypeStruct((_N, _ET, _BT, 8, _CH), jnp.float32),
    scratch_types=[
        pltpu.VMEM((_CPG, _JC, _CH), jnp.int32),
        pltpu.VMEM((2, _CH, _EMBED), jnp.float32),
        pltpu.VMEM((_ET, 8, _CH + 1), jnp.float32),
        pltpu.VMEM((_ET, 8, _CH + 1), jnp.float32),
        pltpu.VMEM((_N, _EMBED), jnp.float32),
        pltpu.SemaphoreType.DMA,
        pltpu.SemaphoreType.DMA,
        pltpu.SemaphoreType.DMA,
        pltpu.SemaphoreType.DMA,
    ],
)
def _embed_sc(xT_hbm, tok_hbm, pos_hbm, out_hbm, idx_v, grows_v, obuf0, obuf1,
              pos_v, gsem0, gsem1, osem0, osem1):
    cid = lax.axis_index("c")
    sid = lax.axis_index("s")
    wid = sid * _NC + cid
    grp = wid // _WPG
    sub = wid % _WPG
    n0 = grp * _CPG
    bt0 = sub * _JC

    pltpu.sync_copy(pos_hbm, pos_v)
    pltpu.sync_copy(xT_hbm.at[pl.ds(n0, _CPG), pl.ds(bt0, _JC)], idx_v)

    iota16 = lax.iota(jnp.int32, 16)
    er = [(iota16 + 16 * d) >> 3 for d in range(4)]
    sv = [(iota16 + 16 * d) & 7 for d in range(4)]

    def gsem(b):
        return gsem0 if b == 0 else gsem1

    def osem(b):
        return osem0 if b == 0 else osem1

    def obuf(b):
        return obuf0 if b == 0 else obuf1

    def fire_gather(t, j):
        b = j % 2
        pltpu.async_copy(tok_hbm.at[idx_v.at[t, j]], grows_v.at[b], gsem(b))

    def out_slice(t, j):
        return out_hbm.at[n0 + t, pl.ds(0, _ET), bt0 + j, pl.ds(0, 8)]

    def obuf_src(b):
        return obuf(b).at[pl.ds(0, _ET), pl.ds(0, 8), pl.ds(0, _CH)]

    def slot(t, j, do_outwait, do_fire):
        b = j % 2
        n = n0 + t
        pltpu.make_async_copy(
            tok_hbm.at[idx_v.at[t, j]], grows_v.at[b], gsem(b)
        ).wait()
        if do_outwait:
            j3 = (j - 2) % _JC
            t3 = t - 1 if j < 2 else t
            pltpu.make_async_copy(obuf_src(b), out_slice(t3, j3), osem(b)).wait()
        prow = [pos_v[n, pl.ds(16 * d, 16)] for d in range(4)]

        def add_body(i, c):
            for ii in range(4):
                row = i * 4 + ii
                rsp = iota16 * 0 + row
                for d in range(4):
                    vals = grows_v[b, row, pl.ds(16 * d, 16)] + prow[d]
                    plsc.store_scatter(obuf(b), [er[d], sv[d], rsp], vals)
            return c

        lax.fori_loop(0, _CH // 4, add_body, 0)
        pltpu.async_copy(obuf_src(b), out_slice(t, j), osem(b))
        if do_fire:
            j2 = (j + 2) % _JC
            t2 = t + 1 if j >= _JC - 2 else t
            fire_gather(t2, j2)

    fire_gather(0, 0)
    fire_gather(0, 1)

    for j in range(_JC):
        slot(0, j, do_outwait=(j >= 2), do_fire=True)

    def t_body(t, carry):
        for j in range(_JC):
            slot(t, j, do_outwait=True, do_fire=True)
        return carry

    lax.fori_loop(1, _CPG - 1, t_body, 0)

    for j in range(_JC):
        slot(_CPG - 1, j, do_outwait=True, do_fire=(j < _JC - 2))

    pltpu.make_async_copy(obuf_src(0), out_slice(_CPG - 1, _JC - 2), osem0).wait()
    pltpu.make_async_copy(obuf_src(1), out_slice(_CPG - 1, _JC - 1), osem1).wait()


def kernel(x, tok_table, pos_table):
    xT3 = jnp.transpose(x.astype(jnp.int32)).reshape(_N, _BT, _CH)
    tiled = _embed_sc(xT3, tok_table, pos_table)
    return jnp.transpose(tiled, (2, 4, 0, 1, 3)).reshape(_B, _N, _EMBED)


# speedup vs baseline: 1.5169x; 1.0008x over previous
import functools

import jax
import jax.numpy as jnp
from jax import lax
from jax.experimental import pallas as pl
from jax.experimental.pallas import tpu as pltpu
from jax.experimental.pallas import tpu_sc as plsc

_VOCAB = 1000000
_EMBED = 64
_B = 4096
_N = 200

_NC = 2
_NS = 16
_NW = _NC * _NS
_WPG = 4
_NG = _NW // _WPG
_CPG = _N // _NG
_QB = _B // _WPG
_CH = 128
_JC = _QB // _CH
_ET = _EMBED // 8
_BT = _B // _CH

_mesh = plsc.VectorSubcoreMesh(core_axis_name="c", subcore_axis_name="s")


@functools.partial(
    pl.kernel,
    mesh=_mesh,
    compiler_params=pltpu.CompilerParams(
        use_tc_tiling_on_sc=False, needs_layout_passes=False
    ),
    out_type=jax.ShapeDtypeStruct((_N, _ET, _BT, 8, _CH), jnp.float32),
    scratch_types=[
        pltpu.VMEM((_CPG, _JC, _CH), jnp.int32),
        pltpu.VMEM((4, _CH, _EMBED), jnp.float32),
        pltpu.VMEM((_ET, 8, _CH + 1), jnp.float32),
        pltpu.VMEM((_ET, 8, _CH + 1), jnp.float32),
        pltpu.VMEM((_N, _EMBED), jnp.float32),
        pltpu.SemaphoreType.DMA,
        pltpu.SemaphoreType.DMA,
        pltpu.SemaphoreType.DMA,
        pltpu.SemaphoreType.DMA,
        pltpu.SemaphoreType.DMA,
        pltpu.SemaphoreType.DMA,
    ],
)
def _embed_sc(xT_hbm, tok_hbm, pos_hbm, out_hbm, idx_v, grows_v, obuf0, obuf1,
              pos_v, gsem0, gsem1, gsem2, gsem3, osem0, osem1):
    cid = lax.axis_index("c")
    sid = lax.axis_index("s")
    wid = sid * _NC + cid
    grp = wid // _WPG
    sub = wid % _WPG
    n0 = grp * _CPG
    bt0 = sub * _JC

    pltpu.sync_copy(pos_hbm, pos_v)
    pltpu.sync_copy(xT_hbm.at[pl.ds(n0, _CPG), pl.ds(bt0, _JC)], idx_v)

    iota16 = lax.iota(jnp.int32, 16)
    er = [(iota16 + 16 * d) >> 3 for d in range(4)]
    sv = [(iota16 + 16 * d) & 7 for d in range(4)]

    def gsem(b):
        return (gsem0, gsem1, gsem2, gsem3)[b]

    def osem(b):
        return osem0 if b == 0 else osem1

    def obuf(b):
        return obuf0 if b == 0 else obuf1

    def fire_gather(t, j):
        b = j % 4
        pltpu.async_copy(tok_hbm.at[idx_v.at[t, j]], grows_v.at[b], gsem(b))

    def out_slice(t, j):
        return out_hbm.at[n0 + t, pl.ds(0, _ET), bt0 + j, pl.ds(0, 8)]

    def obuf_src(b):
        return obuf(b).at[pl.ds(0, _ET), pl.ds(0, 8), pl.ds(0, _CH)]

    def slot(t, j, do_outwait, do_fire):
        g = j % 4
        b = j % 2
        n = n0 + t
        pltpu.make_async_copy(
            tok_hbm.at[idx_v.at[t, j]], grows_v.at[g], gsem(g)
        ).wait()
        if do_outwait:
            j3 = (j - 2) % _JC
            t3 = t - 1 if j < 2 else t
            pltpu.make_async_copy(obuf_src(b), out_slice(t3, j3), osem(b)).wait()
        prow = [pos_v[n, pl.ds(16 * d, 16)] for d in range(4)]

        def add_body(i, c):
            for ii in range(4):
                row = i * 4 + ii
                rsp = iota16 * 0 + row
                for d in range(4):
                    vals = grows_v[g, row, pl.ds(16 * d, 16)] + prow[d]
                    plsc.store_scatter(obuf(b), [er[d], sv[d], rsp], vals)
            return c

        lax.fori_loop(0, _CH // 4, add_body, 0)
        pltpu.async_copy(obuf_src(b), out_slice(t, j), osem(b))
        if do_fire:
            j2 = (j + 4) % _JC
            t2 = t + 1 if j >= _JC - 4 else t
            fire_gather(t2, j2)

    for j in range(4):
        fire_gather(0, j)

    for j in range(_JC):
        slot(0, j, do_outwait=(j >= 2), do_fire=True)

    def t_body(t, carry):
        for j in range(_JC):
            slot(t, j, do_outwait=True, do_fire=True)
        return carry

    lax.fori_loop(1, _CPG - 1, t_body, 0)

    for j in range(_JC):
        slot(_CPG - 1, j, do_outwait=True, do_fire=(j < _JC - 4))

    pltpu.make_async_copy(obuf_src(0), out_slice(_CPG - 1, _JC - 2), osem0).wait()
    pltpu.make_async_copy(obuf_src(1), out_slice(_CPG - 1, _JC - 1), osem1).wait()


def kernel(x, tok_table, pos_table):
    xT3 = jnp.transpose(x.astype(jnp.int32)).reshape(_N, _BT, _CH)
    tiled = _embed_sc(xT3, tok_table, pos_table)
    return jnp.transpose(tiled, (2, 4, 0, 1, 3)).reshape(_B, _N, _EMBED)
